# baseline (speedup 1.0000x reference)
import functools
import math

import jax
import jax.numpy as jnp
from jax import lax
from jax.experimental import pallas as pl
from jax.experimental.pallas import tpu as pltpu
from jax.experimental.pallas import tpu_sc as plsc

B = 8
N = 5000
C = 91
NC = 90
NI = B * NC
NP = 5120
NV = NP // 16
M = 100
OS = 128
CAP = 1024
THR = 0.05
CLIP = math.log(1000.0 / 16.0)
NSC = 32
IPW = (NI + NSC - 1) // NSC

_i32 = jnp.int32
_f32 = jnp.float32


def _softmax_body(x_ref, o_ref):
    blk = o_ref.shape[1]
    j = pl.program_id(1)
    x = x_ref[0]
    lane = jax.lax.broadcasted_iota(_i32, (blk, 128), 1)
    row = jax.lax.broadcasted_iota(_i32, (blk, 128), 0) + j * blk
    x = jnp.where(lane < C, x, -1e30)
    m = jnp.max(x, axis=-1, keepdims=True)
    e = jnp.exp(x - m)
    s = jnp.sum(e, axis=-1, keepdims=True)
    p = e / s
    o_ref[0] = jnp.where(row < N, p, 0.0)


def _softmax(raw_scores):
    blk = 512
    return pl.pallas_call(
        _softmax_body,
        grid=(B, NP // blk),
        in_specs=[pl.BlockSpec((1, blk, 128), lambda b, j: (b, j, 0))],
        out_specs=pl.BlockSpec((1, blk, 128), lambda b, j: (b, j, 0)),
        out_shape=jax.ShapeDtypeStruct((B, NP, 128), _f32),
    )(raw_scores)


def _sc_main_body(scoresT, rb_tbl, an_tbl, img_hbm,
                  outs, outb,
                  sbuf, cidx, cs, bidx, bpar, aidx, apar, brows, arows,
                  by0, bx0, by1, bx1, bar, osc, obx, img_v, sem, sem2, sem3):
    wid = lax.axis_index("s") * 2 + lax.axis_index("c")
    iota16 = lax.iota(_i32, 16)
    pltpu.sync_copy(img_hbm, img_v)

    def inst_body(it, _):
        inst = wid + NSC * it

        @pl.when(inst < NI)
        def _():
            b = inst // NC
            c = inst % NC

            @pl.when(it == 0)
            def _():
                pltpu.sync_copy(scoresT.at[inst], sbuf)

            @pl.when(it > 0)
            def _():
                pltpu.make_async_copy(scoresT.at[inst], sbuf, sem2).wait()
                pltpu.make_async_copy(osc.at[pl.ds(0, OS)], outs.at[inst],
                                      sem3).wait()
                pltpu.make_async_copy(obx.at[pl.ds(0, 4 * OS)], outb.at[inst],
                                      sem3).wait()

            def cbody(j, off):
                base = 16 * j
                v = sbuf[pl.ds(base, 16)]
                m = v > THR
                rank = plsc.cumsum(m.astype(_i32))
                pos = jnp.maximum(off + rank - 1, 0)
                lane = base + iota16
                plsc.store_scatter(cidx, [pos], lane, mask=m)
                plsc.store_scatter(cs, [pos], v, mask=m)
                return off + plsc.all_reduce_population_count(m)[0]

            n0 = lax.fori_loop(0, NV, cbody, _i32(0))

            def _refine():
                def bis(_, lh):
                    lo, hi = lh
                    mid = (lo + hi) // 2
                    tv = plsc.bitcast(jnp.full((16,), mid, _i32), _f32)

                    def cnt(j, acc):
                        v = sbuf[pl.ds(16 * j, 16)]
                        return acc + plsc.all_reduce_population_count(v >= tv)[0]

                    cge = lax.fori_loop(0, NV, cnt, _i32(0))
                    big = cge >= 1000
                    return (jnp.where(big, mid, lo), jnp.where(big, hi, mid))

                lo, _hi = lax.fori_loop(
                    0, 31, bis, (_i32(0x3D4CCCCD), _i32(0x40000000)))
                skv = plsc.bitcast(jnp.full((16,), lo, _i32), _f32)

                def cntg(j, acc):
                    v = sbuf[pl.ds(16 * j, 16)]
                    return acc + plsc.all_reduce_population_count(v > skv)[0]

                ngt = lax.fori_loop(0, NV, cntg, _i32(0))
                quota = 1000 - ngt

                def rc(j, st):
                    off, taken = st
                    base = 16 * j
                    v = sbuf[pl.ds(base, 16)]
                    mgt = v > skv
                    meq = v == skv
                    eqrank = plsc.cumsum(meq.astype(_i32))
                    tk = meq & ((taken + eqrank) <= quota)
                    m = mgt | tk
                    rank = plsc.cumsum(m.astype(_i32))
                    pos = jnp.maximum(off + rank - 1, 0)
                    lane = base + iota16
                    plsc.store_scatter(cidx, [pos], lane, mask=m)
                    plsc.store_scatter(cs, [pos], v, mask=m)
                    tkrank = plsc.cumsum(tk.astype(_i32))
                    return (off + rank[15], taken + tkrank[15])

                off, _t = lax.fori_loop(0, NV, rc, (_i32(0), _i32(0)))
                return off

            n_cand = lax.cond(n0 > 1000, _refine, lambda: n0)

            nxt = inst + NSC

            @pl.when(nxt < NI)
            def _():
                pltpu.async_copy(scoresT.at[nxt], sbuf, sem2)

            plsc.store_scatter(cs, [n_cand + iota16],
                               jnp.full((16,), -2.0, _f32),
                               mask=jnp.full((16,), True))
            plsc.store_scatter(cs, [n_cand + 16 + iota16],
                               jnp.full((16,), -2.0, _f32),
                               mask=jnp.full((16,), True))
            nv = (n_cand + 15) // 16
            nv2 = (n_cand + 31) // 32

            rb_base = b * (N * C) + c + 1
            an_base = b * N

            def gidx(j, _):
                base = 16 * j
                nvals = jnp.clip(cidx[pl.ds(base, 16)], 0, N - 1)
                r4b = nvals * C + rb_base
                r4a = nvals + an_base
                bidx[pl.ds(base, 16)] = r4b // 2
                bpar[pl.ds(base, 16)] = (r4b % 2) * 4
                aidx[pl.ds(base, 16)] = r4a // 2
                apar[pl.ds(base, 16)] = (r4a % 2) * 4
                return 0

            lax.fori_loop(0, CAP // 16, gidx, 0)

            for jj in range(CAP // 128):
                @pl.when(jj * 128 < n_cand)
                def _():
                    sl = pl.ds(jj * 128, 128)
                    pltpu.async_copy(rb_tbl.at[bidx.at[sl]], brows.at[sl], sem)
                    pltpu.async_copy(an_tbl.at[aidx.at[sl]], arows.at[sl], sem)
            for jj in range(CAP // 128):
                @pl.when(jj * 128 < n_cand)
                def _():
                    sl = pl.ds(jj * 128, 128)
                    pltpu.make_async_copy(
                        rb_tbl.at[bidx.at[sl]], brows.at[sl], sem).wait()
                    pltpu.make_async_copy(
                        an_tbl.at[aidx.at[sl]], arows.at[sl], sem).wait()

            hv = plsc.load_gather(img_v, [jnp.full((16,), 8 * b, _i32)])
            wv = plsc.load_gather(img_v, [jnp.full((16,), 8 * b + 1, _i32)])

            def dbody(j, _):
                base = 16 * j
                rows = base + iota16
                sl16 = pl.ds(base, 16)
                zb = bpar[sl16]
                za = apar[sl16]
                dy = plsc.load_gather(brows, [rows, zb])
                dx = plsc.load_gather(brows, [rows, zb + 1])
                dh = plsc.load_gather(brows, [rows, zb + 2])
                dw = plsc.load_gather(brows, [rows, zb + 3])
                ay0 = plsc.load_gather(arows, [rows, za])
                ax0 = plsc.load_gather(arows, [rows, za + 1])
                ay1 = plsc.load_gather(arows, [rows, za + 2])
                ax1 = plsc.load_gather(arows, [rows, za + 3])
                dh = jnp.minimum(dh, CLIP)
                dw = jnp.minimum(dw, CLIP)
                ah = ay1 - ay0 + 1.0
                aw = ax1 - ax0 + 1.0
                yc = dy * ah + (ay0 + 0.5 * ah)
                xc = dx * aw + (ax0 + 0.5 * aw)
                h = jnp.exp(dh) * ah
                w = jnp.exp(dw) * aw
                y0 = yc - 0.5 * h
                x0 = xc - 0.5 * w
                y1 = y0 + h - 1.0
                x1 = x0 + w - 1.0
                y0 = jnp.minimum(jnp.maximum(y0, 0.0), hv)
                x0 = jnp.minimum(jnp.maximum(x0, 0.0), wv)
                y1 = jnp.minimum(jnp.maximum(y1, 0.0), hv)
                x1 = jnp.minimum(jnp.maximum(x1, 0.0), wv)
                sl = pl.ds(base, 16)
                by0[sl] = y0
                bx0[sl] = x0
                by1[sl] = y1
                bx1[sl] = x1
                bar[sl] = jnp.maximum(y1 - y0, 0.0) * jnp.maximum(x1 - x0, 0.0)
                return 0

            lax.fori_loop(0, nv, dbody, 0)

            for t in range(OS // 16 + 1):
                osc[pl.ds(16 * t, 16)] = jnp.full((16,), -1.0, _f32)
            for t in range(4 * OS // 16):
                obx[pl.ds(16 * t, 16)] = jnp.zeros((16,), _f32)

            def amax(j, carry):
                vmax, vidx = carry
                base = 16 * j
                v = cs[pl.ds(base, 16)]
                m = v > vmax
                return (jnp.where(m, v, vmax),
                        jnp.where(m, base + iota16, vidx))

            vmax0, vidx0 = lax.fori_loop(
                0, nv, amax,
                (jnp.full((16,), -4.0, _f32), jnp.zeros((16,), _i32)))
            smax0 = jnp.max(vmax0)
            sidx0 = jnp.min(jnp.where(vmax0 == smax0, vidx0, _i32(NP)))

            def nms_cond(st):
                step, _sidx, smax = st
                return (step < M) & (smax > -1.0)

            lane0 = iota16 == 0
            lane4 = iota16 < 4

            def nms_body(st):
                step, sidx, smax = st
                selv = jnp.full((16,), sidx, _i32)
                sy0 = plsc.load_gather(by0, [selv])
                sx0 = plsc.load_gather(bx0, [selv])
                sy1 = plsc.load_gather(by1, [selv])
                sx1 = plsc.load_gather(bx1, [selv])
                sar = plsc.load_gather(bar, [selv])
                plsc.store_scatter(osc, [jnp.full((16,), step, _i32)],
                                   jnp.full((16,), smax, _f32), mask=lane0)
                coords = jnp.where(iota16 == 0, sy0,
                                   jnp.where(iota16 == 1, sx0,
                                             jnp.where(iota16 == 2, sy1, sx1)))
                plsc.store_scatter(obx, [4 * step + iota16], coords, mask=lane4)

                def one(base):
                    sl = pl.ds(base, 16)
                    v = cs[sl]
                    yy1 = jnp.maximum(by0[sl], sy0)
                    xx1 = jnp.maximum(bx0[sl], sx0)
                    yy2 = jnp.minimum(by1[sl], sy1)
                    xx2 = jnp.minimum(bx1[sl], sx1)
                    inter = (jnp.maximum(yy2 - yy1, 0.0)
                             * jnp.maximum(xx2 - xx1, 0.0))
                    union = bar[sl] + sar - inter
                    pos = union > 0.0
                    iou = jnp.where(pos,
                                    inter / jnp.where(pos, union, 1.0), 0.0)
                    lane = base + iota16
                    vn = jnp.where((iou > 0.5) | (lane == sidx), -1.0, v)
                    cs[sl] = vn
                    return vn, lane

                def sup(j, carry):
                    vmaxa, vidxa, vmaxb, vidxb = carry
                    base = 32 * j
                    vna, lanea = one(base)
                    vnb, laneb = one(base + 16)
                    ma = vna > vmaxa
                    mb = vnb > vmaxb
                    return (jnp.where(ma, vna, vmaxa),
                            jnp.where(ma, lanea, vidxa),
                            jnp.where(mb, vnb, vmaxb),
                            jnp.where(mb, laneb, vidxb))

                neg4 = jnp.full((16,), -4.0, _f32)
                zi = jnp.zeros((16,), _i32)
                vmaxa, vidxa, vmaxb, vidxb = lax.fori_loop(
                    0, nv2, sup, (neg4, zi, neg4, zi))
                mgt = vmaxb > vmaxa
                meq = vmaxb == vmaxa
                vmax = jnp.where(mgt, vmaxb, vmaxa)
                vidx = jnp.where(mgt, vidxb,
                                 jnp.where(meq, jnp.minimum(vidxa, vidxb),
                                           vidxa))
                smax2 = jnp.max(vmax)
                sidx2 = jnp.min(jnp.where(vmax == smax2, vidx, _i32(NP)))
                return (step + 1, sidx2, smax2)

            lax.while_loop(nms_cond, nms_body, (_i32(0), sidx0, smax0))

            pltpu.async_copy(osc.at[pl.ds(0, OS)], outs.at[inst], sem3)
            pltpu.async_copy(obx.at[pl.ds(0, 4 * OS)], outb.at[inst], sem3)

        return 0

    lax.fori_loop(0, IPW, inst_body, 0)
    last = wid + 672 + jnp.where(wid < 16, 32, 0)
    pltpu.make_async_copy(osc.at[pl.ds(0, OS)], outs.at[last], sem3).wait()
    pltpu.make_async_copy(obx.at[pl.ds(0, 4 * OS)], outb.at[last], sem3).wait()


def _sc_main(scoresT, rb_tbl, an_tbl, img):
    mesh = plsc.VectorSubcoreMesh(
        core_axis_name="c", subcore_axis_name="s",
        num_cores=2, num_subcores=16)
    f = pl.kernel(
        _sc_main_body,
        compiler_params=pltpu.CompilerParams(
            needs_layout_passes=False, use_tc_tiling_on_sc=False),
        out_type=[jax.ShapeDtypeStruct((NI, OS), _f32),
                  jax.ShapeDtypeStruct((NI, 4 * OS), _f32)],
        mesh=mesh,
        scratch_types=[
            pltpu.VMEM((NP,), _f32),
            pltpu.VMEM((NP + 16,), _i32),
            pltpu.VMEM((NP + 48,), _f32),
            pltpu.VMEM((CAP,), _i32),
            pltpu.VMEM((CAP,), _i32),
            pltpu.VMEM((CAP,), _i32),
            pltpu.VMEM((CAP,), _i32),
            pltpu.VMEM((CAP, 8), _f32),
            pltpu.VMEM((CAP, 8), _f32),
            pltpu.VMEM((CAP + 48,), _f32),
            pltpu.VMEM((CAP + 48,), _f32),
            pltpu.VMEM((CAP + 48,), _f32),
            pltpu.VMEM((CAP + 48,), _f32),
            pltpu.VMEM((CAP + 48,), _f32),
            pltpu.VMEM((OS + 16,), _f32),
            pltpu.VMEM((4 * OS,), _f32),
            pltpu.VMEM((64,), _f32),
            pltpu.SemaphoreType.DMA,
            pltpu.SemaphoreType.DMA,
            pltpu.SemaphoreType.DMA,
        ],
    )
    return f(scoresT, rb_tbl, an_tbl, img)


def _sc_merge_body(s_all, b_all,
                   obox, oscr, ocls, oval,
                   sbuf, bbuf, hsc, hptr, fsc, fcls, fbox, vbuf):
    wid = lax.axis_index("s") * 2 + lax.axis_index("c")
    iota16 = lax.iota(_i32, 16)

    @pl.when(wid < B)
    def _():
        bb = wid
        pltpu.sync_copy(s_all.at[bb], sbuf)
        pltpu.sync_copy(b_all.at[bb], bbuf)

        for k in range(6):
            cls = 16 * k + iota16
            ok = cls < NC
            idx = jnp.minimum(cls, NC - 1) * OS
            v = plsc.load_gather(sbuf, [idx])
            hsc[pl.ds(16 * k, 16)] = jnp.where(ok, v, -3.0)
            hptr[pl.ds(16 * k, 16)] = jnp.zeros((16,), _i32)

        lane0 = iota16 == 0
        lane4 = iota16 < 4

        def step(t, vc):
            vmax = jnp.full((16,), -4.0, _f32)
            vcls = jnp.zeros((16,), _i32)
            for k in range(6):
                v = hsc[pl.ds(16 * k, 16)]
                m = v > vmax
                vmax = jnp.where(m, v, vmax)
                vcls = jnp.where(m, 16 * k + iota16, vcls)
            smax = jnp.max(vmax)
            scls = jnp.min(jnp.where(vmax == smax, vcls, _i32(999)))
            p = plsc.load_gather(hptr, [jnp.full((16,), scls, _i32)])[0]
            gbase = scls * OS + p
            plsc.store_scatter(fsc, [jnp.full((16,), t, _i32)],
                               jnp.full((16,), smax, _f32), mask=lane0)
            plsc.store_scatter(fcls, [jnp.full((16,), t, _i32)],
                               jnp.full((16,), scls, _i32), mask=lane0)
            bv = plsc.load_gather(
                bbuf, [jnp.minimum(4 * gbase + iota16, 4 * NC * OS - 1)])
            plsc.store_scatter(fbox, [4 * t + iota16], bv, mask=lane4)
            p1 = p + 1
            nxt = plsc.load_gather(
                sbuf, [jnp.full((16,), jnp.minimum(gbase + 1, NC * OS - 1), _i32)])
            newhead = jnp.where(p1 < M, nxt[0], -2.0)
            plsc.store_scatter(hsc, [jnp.full((16,), scls, _i32)],
                               jnp.full((16,), newhead, _f32), mask=lane0)
            plsc.store_scatter(hptr, [jnp.full((16,), scls, _i32)],
                               jnp.full((16,), p1, _i32), mask=lane0)
            return vc + jnp.where(smax > -1.0, 1, 0).astype(_i32)

        vc = lax.fori_loop(0, M, step, _i32(0))
        plsc.store_scatter(vbuf, [iota16],
                           jnp.full((16,), vc, _i32), mask=lane0)
        pltpu.sync_copy(fsc.at[pl.ds(0, OS)], oscr.at[bb])
        pltpu.sync_copy(fcls.at[pl.ds(0, OS)], ocls.at[bb])
        pltpu.sync_copy(fbox.at[pl.ds(0, 4 * OS)], obox.at[bb])
        pltpu.sync_copy(vbuf.at[pl.ds(0, 128)], oval.at[bb])


def _sc_merge(s_all, b_all):
    mesh = plsc.VectorSubcoreMesh(
        core_axis_name="c", subcore_axis_name="s",
        num_cores=2, num_subcores=16)
    f = pl.kernel(
        _sc_merge_body,
        compiler_params=pltpu.CompilerParams(needs_layout_passes=False),
        out_type=[jax.ShapeDtypeStruct((B, 4 * OS), _f32),
                  jax.ShapeDtypeStruct((B, OS), _f32),
                  jax.ShapeDtypeStruct((B, OS), _i32),
                  jax.ShapeDtypeStruct((B, 128), _i32)],
        mesh=mesh,
        scratch_types=[
            pltpu.VMEM((NC * OS,), _f32),
            pltpu.VMEM((NC * OS * 4,), _f32),
            pltpu.VMEM((96,), _f32),
            pltpu.VMEM((96,), _i32),
            pltpu.VMEM((OS + 16,), _f32),
            pltpu.VMEM((OS + 16,), _i32),
            pltpu.VMEM((4 * OS,), _f32),
            pltpu.VMEM((128,), _i32),
        ],
    )
    return f(s_all, b_all)


@jax.jit
def kernel(raw_boxes, raw_scores, anchor_boxes, image_shape):
    probs = _softmax(raw_scores)
    scoresT = jnp.transpose(probs[:, :, 1:C], (0, 2, 1)).reshape(NI, NP)
    rb_tbl = raw_boxes.reshape(B * N * C // 2, 8)
    an_tbl = anchor_boxes.reshape(B * N // 2, 8)
    img = jnp.pad(image_shape, ((0, 0), (0, 6))).reshape(64)
    outs, outb = _sc_main(scoresT, rb_tbl, an_tbl, img)
    s_all = outs.reshape(B, NC * OS)
    b_all = outb.reshape(B, NC * OS * 4)
    obox, oscr, ocls, oval = _sc_merge(s_all, b_all)
    return (obox.reshape(B, OS, 4)[:, :M],
            oscr[:, :M],
            ocls[:, :M],
            oval[:, 0])


# speedup vs baseline: 1.0013x; 1.0013x over previous
import functools
import math

import jax
import jax.numpy as jnp
from jax import lax
from jax.experimental import pallas as pl
from jax.experimental.pallas import tpu as pltpu
from jax.experimental.pallas import tpu_sc as plsc

B = 8
N = 5000
C = 91
NC = 90
NI = B * NC
NP = 5120
NV = NP // 16
M = 100
OS = 128
CAP = 1024
THR = 0.05
CLIP = math.log(1000.0 / 16.0)
NSC = 32
IPW = (NI + NSC - 1) // NSC

_i32 = jnp.int32
_f32 = jnp.float32


def _softmax_body(x_ref, o_ref):
    blk = o_ref.shape[1]
    j = pl.program_id(1)
    x = x_ref[0]
    lane = jax.lax.broadcasted_iota(_i32, (blk, 128), 1)
    row = jax.lax.broadcasted_iota(_i32, (blk, 128), 0) + j * blk
    x = jnp.where(lane < C, x, -1e30)
    m = jnp.max(x, axis=-1, keepdims=True)
    e = jnp.exp(x - m)
    s = jnp.sum(e, axis=-1, keepdims=True)
    p = e / s
    o_ref[0] = jnp.where(row < N, p, 0.0)


def _softmax(raw_scores):
    blk = 512
    return pl.pallas_call(
        _softmax_body,
        grid=(B, NP // blk),
        in_specs=[pl.BlockSpec((1, blk, 128), lambda b, j: (b, j, 0))],
        out_specs=pl.BlockSpec((1, blk, 128), lambda b, j: (b, j, 0)),
        out_shape=jax.ShapeDtypeStruct((B, NP, 128), _f32),
    )(raw_scores)


def _sc_main_body(scoresT, rb_tbl, an_tbl, img_hbm,
                  outs, outb,
                  sbuf, cidx, cs, bidx, bpar, aidx, apar, brows, arows,
                  by0, bx0, by1, bx1, bar, osc, obx, img_v, sem, sem2, sem3):
    wid = lax.axis_index("s") * 2 + lax.axis_index("c")
    iota16 = lax.iota(_i32, 16)
    pltpu.sync_copy(img_hbm, img_v)

    def inst_body(it, _):
        inst = wid + NSC * it

        @pl.when(inst < NI)
        def _():
            b = inst // NC
            c = inst % NC

            @pl.when(it == 0)
            def _():
                pltpu.sync_copy(scoresT.at[inst], sbuf)

            @pl.when(it > 0)
            def _():
                pltpu.make_async_copy(scoresT.at[inst], sbuf, sem2).wait()
                pltpu.make_async_copy(osc.at[pl.ds(0, OS)], outs.at[inst],
                                      sem3).wait()
                pltpu.make_async_copy(obx.at[pl.ds(0, 4 * OS)], outb.at[inst],
                                      sem3).wait()

            def chalf(base, off):
                v = sbuf[pl.ds(base, 16)]
                m = v > THR
                rank = plsc.cumsum(m.astype(_i32))
                pos = jnp.maximum(off + rank - 1, 0)
                lane = base + iota16
                plsc.store_scatter(cidx, [pos], lane, mask=m)
                plsc.store_scatter(cs, [pos], v, mask=m)
                return off + plsc.all_reduce_population_count(m)[0]

            def cbody(j, off):
                base = 32 * j
                off = chalf(base, off)
                return chalf(base + 16, off)

            n0 = lax.fori_loop(0, NV // 2, cbody, _i32(0))

            def _refine():
                def bis(_, lh):
                    lo, hi = lh
                    mid = (lo + hi) // 2
                    tv = plsc.bitcast(jnp.full((16,), mid, _i32), _f32)

                    def cnt(j, acc):
                        v = sbuf[pl.ds(16 * j, 16)]
                        return acc + plsc.all_reduce_population_count(v >= tv)[0]

                    cge = lax.fori_loop(0, NV, cnt, _i32(0))
                    big = cge >= 1000
                    return (jnp.where(big, mid, lo), jnp.where(big, hi, mid))

                lo, _hi = lax.fori_loop(
                    0, 31, bis, (_i32(0x3D4CCCCD), _i32(0x40000000)))
                skv = plsc.bitcast(jnp.full((16,), lo, _i32), _f32)

                def cntg(j, acc):
                    v = sbuf[pl.ds(16 * j, 16)]
                    return acc + plsc.all_reduce_population_count(v > skv)[0]

                ngt = lax.fori_loop(0, NV, cntg, _i32(0))
                quota = 1000 - ngt

                def rc(j, st):
                    off, taken = st
                    base = 16 * j
                    v = sbuf[pl.ds(base, 16)]
                    mgt = v > skv
                    meq = v == skv
                    eqrank = plsc.cumsum(meq.astype(_i32))
                    tk = meq & ((taken + eqrank) <= quota)
                    m = mgt | tk
                    rank = plsc.cumsum(m.astype(_i32))
                    pos = jnp.maximum(off + rank - 1, 0)
                    lane = base + iota16
                    plsc.store_scatter(cidx, [pos], lane, mask=m)
                    plsc.store_scatter(cs, [pos], v, mask=m)
                    tkrank = plsc.cumsum(tk.astype(_i32))
                    return (off + rank[15], taken + tkrank[15])

                off, _t = lax.fori_loop(0, NV, rc, (_i32(0), _i32(0)))
                return off

            n_cand = lax.cond(n0 > 1000, _refine, lambda: n0)

            nxt = inst + NSC

            @pl.when(nxt < NI)
            def _():
                pltpu.async_copy(scoresT.at[nxt], sbuf, sem2)

            plsc.store_scatter(cs, [n_cand + iota16],
                               jnp.full((16,), -2.0, _f32),
                               mask=jnp.full((16,), True))
            plsc.store_scatter(cs, [n_cand + 16 + iota16],
                               jnp.full((16,), -2.0, _f32),
                               mask=jnp.full((16,), True))
            nv = (n_cand + 15) // 16
            nv2 = (n_cand + 31) // 32

            rb_base = b * (N * C) + c + 1
            an_base = b * N

            def gidx(j, _):
                base = 16 * j
                nvals = jnp.clip(cidx[pl.ds(base, 16)], 0, N - 1)
                r4b = nvals * C + rb_base
                r4a = nvals + an_base
                bidx[pl.ds(base, 16)] = r4b // 2
                bpar[pl.ds(base, 16)] = (r4b % 2) * 4
                aidx[pl.ds(base, 16)] = r4a // 2
                apar[pl.ds(base, 16)] = (r4a % 2) * 4
                return 0

            lax.fori_loop(0, CAP // 16, gidx, 0)

            for jj in range(CAP // 128):
                @pl.when(jj * 128 < n_cand)
                def _():
                    sl = pl.ds(jj * 128, 128)
                    pltpu.async_copy(rb_tbl.at[bidx.at[sl]], brows.at[sl], sem)
                    pltpu.async_copy(an_tbl.at[aidx.at[sl]], arows.at[sl], sem)
            for jj in range(CAP // 128):
                @pl.when(jj * 128 < n_cand)
                def _():
                    sl = pl.ds(jj * 128, 128)
                    pltpu.make_async_copy(
                        rb_tbl.at[bidx.at[sl]], brows.at[sl], sem).wait()
                    pltpu.make_async_copy(
                        an_tbl.at[aidx.at[sl]], arows.at[sl], sem).wait()

            hv = plsc.load_gather(img_v, [jnp.full((16,), 8 * b, _i32)])
            wv = plsc.load_gather(img_v, [jnp.full((16,), 8 * b + 1, _i32)])

            def dbody(j, _):
                base = 16 * j
                rows = base + iota16
                sl16 = pl.ds(base, 16)
                zb = bpar[sl16]
                za = apar[sl16]
                dy = plsc.load_gather(brows, [rows, zb])
                dx = plsc.load_gather(brows, [rows, zb + 1])
                dh = plsc.load_gather(brows, [rows, zb + 2])
                dw = plsc.load_gather(brows, [rows, zb + 3])
                ay0 = plsc.load_gather(arows, [rows, za])
                ax0 = plsc.load_gather(arows, [rows, za + 1])
                ay1 = plsc.load_gather(arows, [rows, za + 2])
                ax1 = plsc.load_gather(arows, [rows, za + 3])
                dh = jnp.minimum(dh, CLIP)
                dw = jnp.minimum(dw, CLIP)
                ah = ay1 - ay0 + 1.0
                aw = ax1 - ax0 + 1.0
                yc = dy * ah + (ay0 + 0.5 * ah)
                xc = dx * aw + (ax0 + 0.5 * aw)
                h = jnp.exp(dh) * ah
                w = jnp.exp(dw) * aw
                y0 = yc - 0.5 * h
                x0 = xc - 0.5 * w
                y1 = y0 + h - 1.0
                x1 = x0 + w - 1.0
                y0 = jnp.minimum(jnp.maximum(y0, 0.0), hv)
                x0 = jnp.minimum(jnp.maximum(x0, 0.0), wv)
                y1 = jnp.minimum(jnp.maximum(y1, 0.0), hv)
                x1 = jnp.minimum(jnp.maximum(x1, 0.0), wv)
                sl = pl.ds(base, 16)
                by0[sl] = y0
                bx0[sl] = x0
                by1[sl] = y1
                bx1[sl] = x1
                bar[sl] = jnp.maximum(y1 - y0, 0.0) * jnp.maximum(x1 - x0, 0.0)
                return 0

            lax.fori_loop(0, nv, dbody, 0)

            for t in range(OS // 16 + 1):
                osc[pl.ds(16 * t, 16)] = jnp.full((16,), -1.0, _f32)
            for t in range(4 * OS // 16):
                obx[pl.ds(16 * t, 16)] = jnp.zeros((16,), _f32)

            def amax(j, carry):
                vmax, vidx = carry
                base = 16 * j
                v = cs[pl.ds(base, 16)]
                m = v > vmax
                return (jnp.where(m, v, vmax),
                        jnp.where(m, base + iota16, vidx))

            vmax0, vidx0 = lax.fori_loop(
                0, nv, amax,
                (jnp.full((16,), -4.0, _f32), jnp.zeros((16,), _i32)))
            smax0 = jnp.max(vmax0)
            sidx0 = jnp.min(jnp.where(vmax0 == smax0, vidx0, _i32(NP)))

            def nms_cond(st):
                step, _sidx, smax = st
                return (step < M) & (smax > -1.0)

            lane0 = iota16 == 0
            lane4 = iota16 < 4

            def nms_body(st):
                step, sidx, smax = st
                selv = jnp.full((16,), sidx, _i32)
                sy0 = plsc.load_gather(by0, [selv])
                sx0 = plsc.load_gather(bx0, [selv])
                sy1 = plsc.load_gather(by1, [selv])
                sx1 = plsc.load_gather(bx1, [selv])
                sar = plsc.load_gather(bar, [selv])
                plsc.store_scatter(osc, [jnp.full((16,), step, _i32)],
                                   jnp.full((16,), smax, _f32), mask=lane0)
                coords = jnp.where(iota16 == 0, sy0,
                                   jnp.where(iota16 == 1, sx0,
                                             jnp.where(iota16 == 2, sy1, sx1)))
                plsc.store_scatter(obx, [4 * step + iota16], coords, mask=lane4)

                def one(base):
                    sl = pl.ds(base, 16)
                    v = cs[sl]
                    yy1 = jnp.maximum(by0[sl], sy0)
                    xx1 = jnp.maximum(bx0[sl], sx0)
                    yy2 = jnp.minimum(by1[sl], sy1)
                    xx2 = jnp.minimum(bx1[sl], sx1)
                    inter = (jnp.maximum(yy2 - yy1, 0.0)
                             * jnp.maximum(xx2 - xx1, 0.0))
                    union = bar[sl] + sar - inter
                    pos = union > 0.0
                    iou = jnp.where(pos,
                                    inter / jnp.where(pos, union, 1.0), 0.0)
                    lane = base + iota16
                    vn = jnp.where((iou > 0.5) | (lane == sidx), -1.0, v)
                    cs[sl] = vn
                    return vn, lane

                def sup(j, carry):
                    vmaxa, vidxa, vmaxb, vidxb = carry
                    base = 32 * j
                    vna, lanea = one(base)
                    vnb, laneb = one(base + 16)
                    ma = vna > vmaxa
                    mb = vnb > vmaxb
                    return (jnp.where(ma, vna, vmaxa),
                            jnp.where(ma, lanea, vidxa),
                            jnp.where(mb, vnb, vmaxb),
                            jnp.where(mb, laneb, vidxb))

                neg4 = jnp.full((16,), -4.0, _f32)
                zi = jnp.zeros((16,), _i32)
                vmaxa, vidxa, vmaxb, vidxb = lax.fori_loop(
                    0, nv2, sup, (neg4, zi, neg4, zi))
                mgt = vmaxb > vmaxa
                meq = vmaxb == vmaxa
                vmax = jnp.where(mgt, vmaxb, vmaxa)
                vidx = jnp.where(mgt, vidxb,
                                 jnp.where(meq, jnp.minimum(vidxa, vidxb),
                                           vidxa))
                smax2 = jnp.max(vmax)
                sidx2 = jnp.min(jnp.where(vmax == smax2, vidx, _i32(NP)))
                return (step + 1, sidx2, smax2)

            lax.while_loop(nms_cond, nms_body, (_i32(0), sidx0, smax0))

            pltpu.async_copy(osc.at[pl.ds(0, OS)], outs.at[inst], sem3)
            pltpu.async_copy(obx.at[pl.ds(0, 4 * OS)], outb.at[inst], sem3)

        return 0

    lax.fori_loop(0, IPW, inst_body, 0)
    last = wid + 672 + jnp.where(wid < 16, 32, 0)
    pltpu.make_async_copy(osc.at[pl.ds(0, OS)], outs.at[last], sem3).wait()
    pltpu.make_async_copy(obx.at[pl.ds(0, 4 * OS)], outb.at[last], sem3).wait()


def _sc_main(scoresT, rb_tbl, an_tbl, img):
    mesh = plsc.VectorSubcoreMesh(
        core_axis_name="c", subcore_axis_name="s",
        num_cores=2, num_subcores=16)
    f = pl.kernel(
        _sc_main_body,
        compiler_params=pltpu.CompilerParams(
            needs_layout_passes=False, use_tc_tiling_on_sc=False),
        out_type=[jax.ShapeDtypeStruct((NI, OS), _f32),
                  jax.ShapeDtypeStruct((NI, 4 * OS), _f32)],
        mesh=mesh,
        scratch_types=[
            pltpu.VMEM((NP,), _f32),
            pltpu.VMEM((NP + 16,), _i32),
            pltpu.VMEM((NP + 48,), _f32),
            pltpu.VMEM((CAP,), _i32),
            pltpu.VMEM((CAP,), _i32),
            pltpu.VMEM((CAP,), _i32),
            pltpu.VMEM((CAP,), _i32),
            pltpu.VMEM((CAP, 8), _f32),
            pltpu.VMEM((CAP, 8), _f32),
            pltpu.VMEM((CAP + 48,), _f32),
            pltpu.VMEM((CAP + 48,), _f32),
            pltpu.VMEM((CAP + 48,), _f32),
            pltpu.VMEM((CAP + 48,), _f32),
            pltpu.VMEM((CAP + 48,), _f32),
            pltpu.VMEM((OS + 16,), _f32),
            pltpu.VMEM((4 * OS,), _f32),
            pltpu.VMEM((64,), _f32),
            pltpu.SemaphoreType.DMA,
            pltpu.SemaphoreType.DMA,
            pltpu.SemaphoreType.DMA,
        ],
    )
    return f(scoresT, rb_tbl, an_tbl, img)


def _sc_merge_body(s_all, b_all,
                   obox, oscr, ocls, oval,
                   sbuf, bbuf, hsc, hptr, fsc, fcls, fbox, vbuf):
    wid = lax.axis_index("s") * 2 + lax.axis_index("c")
    iota16 = lax.iota(_i32, 16)

    @pl.when(wid < B)
    def _():
        bb = wid
        pltpu.sync_copy(s_all.at[bb], sbuf)
        pltpu.sync_copy(b_all.at[bb], bbuf)

        for k in range(6):
            cls = 16 * k + iota16
            ok = cls < NC
            idx = jnp.minimum(cls, NC - 1) * OS
            v = plsc.load_gather(sbuf, [idx])
            hsc[pl.ds(16 * k, 16)] = jnp.where(ok, v, -3.0)
            hptr[pl.ds(16 * k, 16)] = jnp.zeros((16,), _i32)

        lane0 = iota16 == 0
        lane4 = iota16 < 4

        def step(t, vc):
            vmax = jnp.full((16,), -4.0, _f32)
            vcls = jnp.zeros((16,), _i32)
            for k in range(6):
                v = hsc[pl.ds(16 * k, 16)]
                m = v > vmax
                vmax = jnp.where(m, v, vmax)
                vcls = jnp.where(m, 16 * k + iota16, vcls)
            smax = jnp.max(vmax)
            scls = jnp.min(jnp.where(vmax == smax, vcls, _i32(999)))
            p = plsc.load_gather(hptr, [jnp.full((16,), scls, _i32)])[0]
            gbase = scls * OS + p
            plsc.store_scatter(fsc, [jnp.full((16,), t, _i32)],
                               jnp.full((16,), smax, _f32), mask=lane0)
            plsc.store_scatter(fcls, [jnp.full((16,), t, _i32)],
                               jnp.full((16,), scls, _i32), mask=lane0)
            bv = plsc.load_gather(
                bbuf, [jnp.minimum(4 * gbase + iota16, 4 * NC * OS - 1)])
            plsc.store_scatter(fbox, [4 * t + iota16], bv, mask=lane4)
            p1 = p + 1
            nxt = plsc.load_gather(
                sbuf, [jnp.full((16,), jnp.minimum(gbase + 1, NC * OS - 1), _i32)])
            newhead = jnp.where(p1 < M, nxt[0], -2.0)
            plsc.store_scatter(hsc, [jnp.full((16,), scls, _i32)],
                               jnp.full((16,), newhead, _f32), mask=lane0)
            plsc.store_scatter(hptr, [jnp.full((16,), scls, _i32)],
                               jnp.full((16,), p1, _i32), mask=lane0)
            return vc + jnp.where(smax > -1.0, 1, 0).astype(_i32)

        vc = lax.fori_loop(0, M, step, _i32(0))
        plsc.store_scatter(vbuf, [iota16],
                           jnp.full((16,), vc, _i32), mask=lane0)
        pltpu.sync_copy(fsc.at[pl.ds(0, OS)], oscr.at[bb])
        pltpu.sync_copy(fcls.at[pl.ds(0, OS)], ocls.at[bb])
        pltpu.sync_copy(fbox.at[pl.ds(0, 4 * OS)], obox.at[bb])
        pltpu.sync_copy(vbuf.at[pl.ds(0, 128)], oval.at[bb])


def _sc_merge(s_all, b_all):
    mesh = plsc.VectorSubcoreMesh(
        core_axis_name="c", subcore_axis_name="s",
        num_cores=2, num_subcores=16)
    f = pl.kernel(
        _sc_merge_body,
        compiler_params=pltpu.CompilerParams(needs_layout_passes=False),
        out_type=[jax.ShapeDtypeStruct((B, 4 * OS), _f32),
                  jax.ShapeDtypeStruct((B, OS), _f32),
                  jax.ShapeDtypeStruct((B, OS), _i32),
                  jax.ShapeDtypeStruct((B, 128), _i32)],
        mesh=mesh,
        scratch_types=[
            pltpu.VMEM((NC * OS,), _f32),
            pltpu.VMEM((NC * OS * 4,), _f32),
            pltpu.VMEM((96,), _f32),
            pltpu.VMEM((96,), _i32),
            pltpu.VMEM((OS + 16,), _f32),
            pltpu.VMEM((OS + 16,), _i32),
            pltpu.VMEM((4 * OS,), _f32),
            pltpu.VMEM((128,), _i32),
        ],
    )
    return f(s_all, b_all)


@jax.jit
def kernel(raw_boxes, raw_scores, anchor_boxes, image_shape):
    probs = _softmax(raw_scores)
    scoresT = jnp.transpose(probs[:, :, 1:C], (0, 2, 1)).reshape(NI, NP)
    rb_tbl = raw_boxes.reshape(B * N * C // 2, 8)
    an_tbl = anchor_boxes.reshape(B * N // 2, 8)
    img = jnp.pad(image_shape, ((0, 0), (0, 6))).reshape(64)
    outs, outb = _sc_main(scoresT, rb_tbl, an_tbl, img)
    s_all = outs.reshape(B, NC * OS)
    b_all = outb.reshape(B, NC * OS * 4)
    obox, oscr, ocls, oval = _sc_merge(s_all, b_all)
    return (obox.reshape(B, OS, 4)[:, :M],
            oscr[:, :M],
            ocls[:, :M],
            oval[:, 0])


# speedup vs baseline: 1.0708x; 1.0694x over previous
import functools
import math

import jax
import jax.numpy as jnp
from jax import lax
from jax.experimental import pallas as pl
from jax.experimental.pallas import tpu as pltpu
from jax.experimental.pallas import tpu_sc as plsc

B = 8
N = 5000
C = 91
NC = 90
NI = B * NC
NP = 5120
NV = NP // 16
M = 100
OS = 128
CAP = 1024
THR = 0.05
CLIP = math.log(1000.0 / 16.0)
NSC = 32
IPW = (NI + NSC - 1) // NSC

_i32 = jnp.int32
_f32 = jnp.float32


def _softmax_body(x_ref, o_ref):
    blk = x_ref.shape[1]
    j = pl.program_id(1)
    x = x_ref[0]
    lane = jax.lax.broadcasted_iota(_i32, (blk, 128), 1)
    row = jax.lax.broadcasted_iota(_i32, (blk, 128), 0) + j * blk
    x = jnp.where(lane < C, x, -1e30)
    m = jnp.max(x, axis=-1, keepdims=True)
    e = jnp.exp(x - m)
    s = jnp.sum(e, axis=-1, keepdims=True)
    p = e / s
    o_ref[...] = jnp.transpose(jnp.where(row < N, p, 0.0), (1, 0))


def _softmax(raw_scores):
    blk = 512
    return pl.pallas_call(
        _softmax_body,
        grid=(B, NP // blk),
        in_specs=[pl.BlockSpec((1, blk, 128), lambda b, j: (b, j, 0))],
        out_specs=pl.BlockSpec((128, blk), lambda b, j: (b, j)),
        out_shape=jax.ShapeDtypeStruct((B * 128, NP), _f32),
    )(raw_scores)


def _sc_main_body(scoresT, rb_tbl, an_tbl, img_hbm,
                  outs, outb,
                  sbuf, cidx, cs, bidx, bpar, aidx, apar, brows, arows,
                  by0, bx0, by1, bx1, bar, osc, obx, img_v, sem, sem2, sem3):
    wid = lax.axis_index("s") * 2 + lax.axis_index("c")
    iota16 = lax.iota(_i32, 16)
    pltpu.sync_copy(img_hbm, img_v)

    def inst_body(it, _):
        inst = wid + NSC * it

        @pl.when(inst < NI)
        def _():
            b = inst // NC
            c = inst % NC

            @pl.when(it == 0)
            def _():
                pltpu.sync_copy(scoresT.at[b * 128 + c + 1], sbuf)

            @pl.when(it > 0)
            def _():
                pltpu.make_async_copy(
                    scoresT.at[b * 128 + c + 1], sbuf, sem2).wait()
                pltpu.make_async_copy(osc.at[pl.ds(0, OS)], outs.at[inst],
                                      sem3).wait()
                pltpu.make_async_copy(obx.at[pl.ds(0, 4 * OS)], outb.at[inst],
                                      sem3).wait()

            def chalf(base, off):
                v = sbuf[pl.ds(base, 16)]
                m = v > THR
                rank = plsc.cumsum(m.astype(_i32))
                pos = jnp.maximum(off + rank - 1, 0)
                lane = base + iota16
                plsc.store_scatter(cidx, [pos], lane, mask=m)
                plsc.store_scatter(cs, [pos], v, mask=m)
                return off + plsc.all_reduce_population_count(m)[0]

            def cbody(j, off):
                base = 32 * j
                off = chalf(base, off)
                return chalf(base + 16, off)

            n0 = lax.fori_loop(0, NV // 2, cbody, _i32(0))

            def _refine():
                def bis(_, lh):
                    lo, hi = lh
                    mid = (lo + hi) // 2
                    tv = plsc.bitcast(jnp.full((16,), mid, _i32), _f32)

                    def cnt(j, acc):
                        v = sbuf[pl.ds(16 * j, 16)]
                        return acc + plsc.all_reduce_population_count(v >= tv)[0]

                    cge = lax.fori_loop(0, NV, cnt, _i32(0))
                    big = cge >= 1000
                    return (jnp.where(big, mid, lo), jnp.where(big, hi, mid))

                lo, _hi = lax.fori_loop(
                    0, 31, bis, (_i32(0x3D4CCCCD), _i32(0x40000000)))
                skv = plsc.bitcast(jnp.full((16,), lo, _i32), _f32)

                def cntg(j, acc):
                    v = sbuf[pl.ds(16 * j, 16)]
                    return acc + plsc.all_reduce_population_count(v > skv)[0]

                ngt = lax.fori_loop(0, NV, cntg, _i32(0))
                quota = 1000 - ngt

                def rc(j, st):
                    off, taken = st
                    base = 16 * j
                    v = sbuf[pl.ds(base, 16)]
                    mgt = v > skv
                    meq = v == skv
                    eqrank = plsc.cumsum(meq.astype(_i32))
                    tk = meq & ((taken + eqrank) <= quota)
                    m = mgt | tk
                    rank = plsc.cumsum(m.astype(_i32))
                    pos = jnp.maximum(off + rank - 1, 0)
                    lane = base + iota16
                    plsc.store_scatter(cidx, [pos], lane, mask=m)
                    plsc.store_scatter(cs, [pos], v, mask=m)
                    tkrank = plsc.cumsum(tk.astype(_i32))
                    return (off + rank[15], taken + tkrank[15])

                off, _t = lax.fori_loop(0, NV, rc, (_i32(0), _i32(0)))
                return off

            n_cand = lax.cond(n0 > 1000, _refine, lambda: n0)

            nxt = inst + NSC

            @pl.when(nxt < NI)
            def _():
                pltpu.async_copy(
                    scoresT.at[(nxt // NC) * 128 + nxt % NC + 1], sbuf, sem2)

            plsc.store_scatter(cs, [n_cand + iota16],
                               jnp.full((16,), -2.0, _f32),
                               mask=jnp.full((16,), True))
            plsc.store_scatter(cs, [n_cand + 16 + iota16],
                               jnp.full((16,), -2.0, _f32),
                               mask=jnp.full((16,), True))
            nv = (n_cand + 15) // 16
            nv2 = (n_cand + 31) // 32

            rb_base = b * (N * C) + c + 1
            an_base = b * N

            def gidx(j, _):
                base = 16 * j
                nvals = jnp.clip(cidx[pl.ds(base, 16)], 0, N - 1)
                r4b = nvals * C + rb_base
                r4a = nvals + an_base
                bidx[pl.ds(base, 16)] = r4b // 2
                bpar[pl.ds(base, 16)] = (r4b % 2) * 4
                aidx[pl.ds(base, 16)] = r4a // 2
                apar[pl.ds(base, 16)] = (r4a % 2) * 4
                return 0

            lax.fori_loop(0, CAP // 16, gidx, 0)

            for jj in range(CAP // 128):
                @pl.when(jj * 128 < n_cand)
                def _():
                    sl = pl.ds(jj * 128, 128)
                    pltpu.async_copy(rb_tbl.at[bidx.at[sl]], brows.at[sl], sem)
                    pltpu.async_copy(an_tbl.at[aidx.at[sl]], arows.at[sl], sem)
            for jj in range(CAP // 128):
                @pl.when(jj * 128 < n_cand)
                def _():
                    sl = pl.ds(jj * 128, 128)
                    pltpu.make_async_copy(
                        rb_tbl.at[bidx.at[sl]], brows.at[sl], sem).wait()
                    pltpu.make_async_copy(
                        an_tbl.at[aidx.at[sl]], arows.at[sl], sem).wait()

            hv = plsc.load_gather(img_v, [jnp.full((16,), 8 * b, _i32)])
            wv = plsc.load_gather(img_v, [jnp.full((16,), 8 * b + 1, _i32)])

            def dbody(j, _):
                base = 16 * j
                rows = base + iota16
                sl16 = pl.ds(base, 16)
                zb = bpar[sl16]
                za = apar[sl16]
                dy = plsc.load_gather(brows, [rows, zb])
                dx = plsc.load_gather(brows, [rows, zb + 1])
                dh = plsc.load_gather(brows, [rows, zb + 2])
                dw = plsc.load_gather(brows, [rows, zb + 3])
                ay0 = plsc.load_gather(arows, [rows, za])
                ax0 = plsc.load_gather(arows, [rows, za + 1])
                ay1 = plsc.load_gather(arows, [rows, za + 2])
                ax1 = plsc.load_gather(arows, [rows, za + 3])
                dh = jnp.minimum(dh, CLIP)
                dw = jnp.minimum(dw, CLIP)
                ah = ay1 - ay0 + 1.0
                aw = ax1 - ax0 + 1.0
                yc = dy * ah + (ay0 + 0.5 * ah)
                xc = dx * aw + (ax0 + 0.5 * aw)
                h = jnp.exp(dh) * ah
                w = jnp.exp(dw) * aw
                y0 = yc - 0.5 * h
                x0 = xc - 0.5 * w
                y1 = y0 + h - 1.0
                x1 = x0 + w - 1.0
                y0 = jnp.minimum(jnp.maximum(y0, 0.0), hv)
                x0 = jnp.minimum(jnp.maximum(x0, 0.0), wv)
                y1 = jnp.minimum(jnp.maximum(y1, 0.0), hv)
                x1 = jnp.minimum(jnp.maximum(x1, 0.0), wv)
                sl = pl.ds(base, 16)
                by0[sl] = y0
                bx0[sl] = x0
                by1[sl] = y1
                bx1[sl] = x1
                bar[sl] = jnp.maximum(y1 - y0, 0.0) * jnp.maximum(x1 - x0, 0.0)
                return 0

            lax.fori_loop(0, nv, dbody, 0)

            for t in range(OS // 16 + 1):
                osc[pl.ds(16 * t, 16)] = jnp.full((16,), -1.0, _f32)
            for t in range(4 * OS // 16):
                obx[pl.ds(16 * t, 16)] = jnp.zeros((16,), _f32)

            def amax(j, carry):
                vmax, vidx = carry
                base = 16 * j
                v = cs[pl.ds(base, 16)]
                m = v > vmax
                return (jnp.where(m, v, vmax),
                        jnp.where(m, base + iota16, vidx))

            vmax0, vidx0 = lax.fori_loop(
                0, nv, amax,
                (jnp.full((16,), -4.0, _f32), jnp.zeros((16,), _i32)))
            smax0 = jnp.max(vmax0)
            sidx0 = jnp.min(jnp.where(vmax0 == smax0, vidx0, _i32(NP)))

            def nms_cond(st):
                step, _sidx, smax = st
                return (step < M) & (smax > -1.0)

            lane0 = iota16 == 0
            lane4 = iota16 < 4

            def nms_body(st):
                step, sidx, smax = st
                selv = jnp.full((16,), sidx, _i32)
                sy0 = plsc.load_gather(by0, [selv])
                sx0 = plsc.load_gather(bx0, [selv])
                sy1 = plsc.load_gather(by1, [selv])
                sx1 = plsc.load_gather(bx1, [selv])
                sar = plsc.load_gather(bar, [selv])
                plsc.store_scatter(osc, [jnp.full((16,), step, _i32)],
                                   jnp.full((16,), smax, _f32), mask=lane0)
                coords = jnp.where(iota16 == 0, sy0,
                                   jnp.where(iota16 == 1, sx0,
                                             jnp.where(iota16 == 2, sy1, sx1)))
                plsc.store_scatter(obx, [4 * step + iota16], coords, mask=lane4)

                def one(base):
                    sl = pl.ds(base, 16)
                    v = cs[sl]
                    yy1 = jnp.maximum(by0[sl], sy0)
                    xx1 = jnp.maximum(bx0[sl], sx0)
                    yy2 = jnp.minimum(by1[sl], sy1)
                    xx2 = jnp.minimum(bx1[sl], sx1)
                    inter = (jnp.maximum(yy2 - yy1, 0.0)
                             * jnp.maximum(xx2 - xx1, 0.0))
                    union = bar[sl] + sar - inter
                    pos = union > 0.0
                    iou = jnp.where(pos,
                                    inter / jnp.where(pos, union, 1.0), 0.0)
                    lane = base + iota16
                    vn = jnp.where((iou > 0.5) | (lane == sidx), -1.0, v)
                    cs[sl] = vn
                    return vn, lane

                def sup(j, carry):
                    vmaxa, vidxa, vmaxb, vidxb = carry
                    base = 32 * j
                    vna, lanea = one(base)
                    vnb, laneb = one(base + 16)
                    ma = vna > vmaxa
                    mb = vnb > vmaxb
                    return (jnp.where(ma, vna, vmaxa),
                            jnp.where(ma, lanea, vidxa),
                            jnp.where(mb, vnb, vmaxb),
                            jnp.where(mb, laneb, vidxb))

                neg4 = jnp.full((16,), -4.0, _f32)
                zi = jnp.zeros((16,), _i32)
                vmaxa, vidxa, vmaxb, vidxb = lax.fori_loop(
                    0, nv2, sup, (neg4, zi, neg4, zi))
                mgt = vmaxb > vmaxa
                meq = vmaxb == vmaxa
                vmax = jnp.where(mgt, vmaxb, vmaxa)
                vidx = jnp.where(mgt, vidxb,
                                 jnp.where(meq, jnp.minimum(vidxa, vidxb),
                                           vidxa))
                smax2 = jnp.max(vmax)
                sidx2 = jnp.min(jnp.where(vmax == smax2, vidx, _i32(NP)))
                return (step + 1, sidx2, smax2)

            lax.while_loop(nms_cond, nms_body, (_i32(0), sidx0, smax0))

            pltpu.async_copy(osc.at[pl.ds(0, OS)], outs.at[inst], sem3)
            pltpu.async_copy(obx.at[pl.ds(0, 4 * OS)], outb.at[inst], sem3)

        return 0

    lax.fori_loop(0, IPW, inst_body, 0)
    last = wid + 672 + jnp.where(wid < 16, 32, 0)
    pltpu.make_async_copy(osc.at[pl.ds(0, OS)], outs.at[last], sem3).wait()
    pltpu.make_async_copy(obx.at[pl.ds(0, 4 * OS)], outb.at[last], sem3).wait()


def _sc_main(scoresT, rb_tbl, an_tbl, img):
    mesh = plsc.VectorSubcoreMesh(
        core_axis_name="c", subcore_axis_name="s",
        num_cores=2, num_subcores=16)
    f = pl.kernel(
        _sc_main_body,
        compiler_params=pltpu.CompilerParams(
            needs_layout_passes=False, use_tc_tiling_on_sc=False),
        out_type=[jax.ShapeDtypeStruct((NI, OS), _f32),
                  jax.ShapeDtypeStruct((NI, 4 * OS), _f32)],
        mesh=mesh,
        scratch_types=[
            pltpu.VMEM((NP,), _f32),
            pltpu.VMEM((NP + 16,), _i32),
            pltpu.VMEM((NP + 48,), _f32),
            pltpu.VMEM((CAP,), _i32),
            pltpu.VMEM((CAP,), _i32),
            pltpu.VMEM((CAP,), _i32),
            pltpu.VMEM((CAP,), _i32),
            pltpu.VMEM((CAP, 8), _f32),
            pltpu.VMEM((CAP, 8), _f32),
            pltpu.VMEM((CAP + 48,), _f32),
            pltpu.VMEM((CAP + 48,), _f32),
            pltpu.VMEM((CAP + 48,), _f32),
            pltpu.VMEM((CAP + 48,), _f32),
            pltpu.VMEM((CAP + 48,), _f32),
            pltpu.VMEM((OS + 16,), _f32),
            pltpu.VMEM((4 * OS,), _f32),
            pltpu.VMEM((64,), _f32),
            pltpu.SemaphoreType.DMA,
            pltpu.SemaphoreType.DMA,
            pltpu.SemaphoreType.DMA,
        ],
    )
    return f(scoresT, rb_tbl, an_tbl, img)


def _sc_merge_body(s_all, b_all,
                   obox, oscr, ocls, oval,
                   sbuf, bbuf, hsc, hptr, fsc, fcls, fbox, vbuf):
    wid = lax.axis_index("s") * 2 + lax.axis_index("c")
    iota16 = lax.iota(_i32, 16)

    @pl.when(wid < B)
    def _():
        bb = wid
        pltpu.sync_copy(s_all.at[bb], sbuf)
        pltpu.sync_copy(b_all.at[bb], bbuf)

        for k in range(6):
            cls = 16 * k + iota16
            ok = cls < NC
            idx = jnp.minimum(cls, NC - 1) * OS
            v = plsc.load_gather(sbuf, [idx])
            hsc[pl.ds(16 * k, 16)] = jnp.where(ok, v, -3.0)
            hptr[pl.ds(16 * k, 16)] = jnp.zeros((16,), _i32)

        lane0 = iota16 == 0
        lane4 = iota16 < 4

        def step(t, vc):
            vmax = jnp.full((16,), -4.0, _f32)
            vcls = jnp.zeros((16,), _i32)
            for k in range(6):
                v = hsc[pl.ds(16 * k, 16)]
                m = v > vmax
                vmax = jnp.where(m, v, vmax)
                vcls = jnp.where(m, 16 * k + iota16, vcls)
            smax = jnp.max(vmax)
            scls = jnp.min(jnp.where(vmax == smax, vcls, _i32(999)))
            p = plsc.load_gather(hptr, [jnp.full((16,), scls, _i32)])[0]
            gbase = scls * OS + p
            plsc.store_scatter(fsc, [jnp.full((16,), t, _i32)],
                               jnp.full((16,), smax, _f32), mask=lane0)
            plsc.store_scatter(fcls, [jnp.full((16,), t, _i32)],
                               jnp.full((16,), scls, _i32), mask=lane0)
            bv = plsc.load_gather(
                bbuf, [jnp.minimum(4 * gbase + iota16, 4 * NC * OS - 1)])
            plsc.store_scatter(fbox, [4 * t + iota16], bv, mask=lane4)
            p1 = p + 1
            nxt = plsc.load_gather(
                sbuf, [jnp.full((16,), jnp.minimum(gbase + 1, NC * OS - 1), _i32)])
            newhead = jnp.where(p1 < M, nxt[0], -2.0)
            plsc.store_scatter(hsc, [jnp.full((16,), scls, _i32)],
                               jnp.full((16,), newhead, _f32), mask=lane0)
            plsc.store_scatter(hptr, [jnp.full((16,), scls, _i32)],
                               jnp.full((16,), p1, _i32), mask=lane0)
            return vc + jnp.where(smax > -1.0, 1, 0).astype(_i32)

        vc = lax.fori_loop(0, M, step, _i32(0))
        plsc.store_scatter(vbuf, [iota16],
                           jnp.full((16,), vc, _i32), mask=lane0)
        pltpu.sync_copy(fsc.at[pl.ds(0, OS)], oscr.at[bb])
        pltpu.sync_copy(fcls.at[pl.ds(0, OS)], ocls.at[bb])
        pltpu.sync_copy(fbox.at[pl.ds(0, 4 * OS)], obox.at[bb])
        pltpu.sync_copy(vbuf.at[pl.ds(0, 128)], oval.at[bb])


def _sc_merge(s_all, b_all):
    mesh = plsc.VectorSubcoreMesh(
        core_axis_name="c", subcore_axis_name="s",
        num_cores=2, num_subcores=16)
    f = pl.kernel(
        _sc_merge_body,
        compiler_params=pltpu.CompilerParams(needs_layout_passes=False),
        out_type=[jax.ShapeDtypeStruct((B, 4 * OS), _f32),
                  jax.ShapeDtypeStruct((B, OS), _f32),
                  jax.ShapeDtypeStruct((B, OS), _i32),
                  jax.ShapeDtypeStruct((B, 128), _i32)],
        mesh=mesh,
        scratch_types=[
            pltpu.VMEM((NC * OS,), _f32),
            pltpu.VMEM((NC * OS * 4,), _f32),
            pltpu.VMEM((96,), _f32),
            pltpu.VMEM((96,), _i32),
            pltpu.VMEM((OS + 16,), _f32),
            pltpu.VMEM((OS + 16,), _i32),
            pltpu.VMEM((4 * OS,), _f32),
            pltpu.VMEM((128,), _i32),
        ],
    )
    return f(s_all, b_all)


@jax.jit
def kernel(raw_boxes, raw_scores, anchor_boxes, image_shape):
    scoresT = _softmax(raw_scores)
    rb_tbl = raw_boxes.reshape(B * N * C // 2, 8)
    an_tbl = anchor_boxes.reshape(B * N // 2, 8)
    img = jnp.pad(image_shape, ((0, 0), (0, 6))).reshape(64)
    outs, outb = _sc_main(scoresT, rb_tbl, an_tbl, img)
    s_all = outs.reshape(B, NC * OS)
    b_all = outb.reshape(B, NC * OS * 4)
    obox, oscr, ocls, oval = _sc_merge(s_all, b_all)
    return (obox.reshape(B, OS, 4)[:, :M],
            oscr[:, :M],
            ocls[:, :M],
            oval[:, 0])
